# baseline (speedup 1.0000x reference)
import functools

import jax
import jax.numpy as jnp
from jax import lax
from jax.experimental import pallas as pl
from jax.experimental.pallas import tpu as pltpu
from jax.experimental.pallas import tpu_sc as plsc

B = 8192
D = 1024
E = 32
K = 3
H = 256
O = 64
O2 = 128
Q = 11399

T = 256
NT = (B * K) // T + E
P = NT * T
TB1 = 512
TB2 = 512
TQ = 512

NW = 32
RPW = P // NW
CH = 64
NCH = RPW // CH
BKW = (B * K) // NW

_F32 = jnp.float32
_I32 = jnp.int32


def _gelu(x):
    return 0.5 * x * (1.0 + lax.erf(x * 0.7071067811865476))


def _router_shared_body(x_ref, rw_ref, rb_ref, w1_ref, b1_ref, w2_ref, b2_ref,
                        idx_ref, w_ref, sh_ref, stats_ref):
    i = pl.program_id(0)
    x = x_ref[...]
    logits = jnp.dot(x, rw_ref[...], preferred_element_type=_F32) + rb_ref[...]
    m = jnp.max(logits, axis=1, keepdims=True)
    ex = jnp.exp(logits - m)
    probs = ex / jnp.sum(ex, axis=1, keepdims=True)

    @pl.when(i == 0)
    def _():
        stats_ref[...] = jnp.zeros_like(stats_ref)

    stats_ref[0:1, 0:E] += jnp.sum(probs, axis=0, keepdims=True)

    iota = lax.broadcasted_iota(_I32, probs.shape, 1)
    pw = probs
    cols_i = []
    cols_w = []
    for _ in range(K):
        mk = jnp.max(pw, axis=1, keepdims=True)
        a = jnp.min(jnp.where(pw == mk, iota, E), axis=1, keepdims=True)
        cols_i.append(a)
        cols_w.append(mk)
        pw = jnp.where(iota == a, -1.0, pw)
    zi = jnp.zeros_like(cols_i[0])
    zw = jnp.zeros_like(cols_w[0])
    idx_full = jnp.concatenate(cols_i + [zi], axis=1)
    w_full = jnp.concatenate(cols_w + [zw], axis=1)
    wsum = cols_w[0] + cols_w[1] + cols_w[2]
    idx_ref[...] = idx_full
    w_ref[...] = w_full / wsum

    mid = _gelu(jnp.dot(x, w1_ref[...], preferred_element_type=_F32) + b1_ref[...])
    sh_ref[...] = jnp.dot(mid, w2_ref[...], preferred_element_type=_F32) + b2_ref[...]


def _router_shared(x, router_W, router_b, sh_W1, sh_b1, sh_W2, sh_b2):
    grid = (B // TB1,)
    return pl.pallas_call(
        _router_shared_body,
        grid=grid,
        in_specs=[
            pl.BlockSpec((TB1, D), lambda i: (i, 0)),
            pl.BlockSpec((D, E), lambda i: (0, 0)),
            pl.BlockSpec((1, E), lambda i: (0, 0)),
            pl.BlockSpec((D, H), lambda i: (0, 0)),
            pl.BlockSpec((1, H), lambda i: (0, 0)),
            pl.BlockSpec((H, O), lambda i: (0, 0)),
            pl.BlockSpec((1, O), lambda i: (0, 0)),
        ],
        out_specs=[
            pl.BlockSpec((TB1, 4), lambda i: (i, 0)),
            pl.BlockSpec((TB1, 4), lambda i: (i, 0)),
            pl.BlockSpec((TB1, O), lambda i: (i, 0)),
            pl.BlockSpec((8, 128), lambda i: (0, 0)),
        ],
        out_shape=[
            jax.ShapeDtypeStruct((B, 4), _I32),
            jax.ShapeDtypeStruct((B, 4), _F32),
            jax.ShapeDtypeStruct((B, O), _F32),
            jax.ShapeDtypeStruct((8, 128), _F32),
        ],
    )(x, router_W, router_b.reshape(1, E), sh_W1, sh_b1.reshape(1, H),
      sh_W2, sh_b2.reshape(1, O))


@functools.cache
def _make_sc_gather(n_rows, width, chunk):
    nch = n_rows // NW // chunk
    mesh = plsc.VectorSubcoreMesh(core_axis_name="c", subcore_axis_name="s")

    @functools.partial(
        pl.kernel,
        mesh=mesh,
        out_type=jax.ShapeDtypeStruct((n_rows, width), _F32),
        scratch_types=[
            pltpu.VMEM((chunk,), _I32),
            pltpu.VMEM((chunk, width), _F32),
            pltpu.SemaphoreType.DMA,
        ],
    )
    def sc_gather(src_hbm, idx_hbm, out_hbm, idx_v, rows_v, sem):
        wid = lax.axis_index("s") * 2 + lax.axis_index("c")
        base = wid * (n_rows // NW)
        for c in range(nch):
            off = base + c * chunk
            pltpu.sync_copy(idx_hbm.at[pl.ds(off, chunk)], idx_v)
            pltpu.async_copy(src_hbm.at[idx_v], rows_v, sem).wait()
            pltpu.sync_copy(rows_v, out_hbm.at[pl.ds(off, chunk)])

    return sc_gather


def _expert_mlp_body(te_ref, xs_ref, w1_ref, b1_ref, w2_ref, b2_ref, out_ref):
    h = _gelu(jnp.dot(xs_ref[...], w1_ref[0], preferred_element_type=_F32)
              + b1_ref[0])
    out_ref[...] = jnp.dot(h, w2_ref[0], preferred_element_type=_F32) + b2_ref[0]


def _expert_mlp(x_sorted, tile_expert, ex_W1, ex_b1, ex_W2, ex_b2):
    grid_spec = pltpu.PrefetchScalarGridSpec(
        num_scalar_prefetch=1,
        grid=(NT,),
        in_specs=[
            pl.BlockSpec((T, D), lambda t, te: (t, 0)),
            pl.BlockSpec((1, D, H), lambda t, te: (te[t], 0, 0)),
            pl.BlockSpec((1, 1, H), lambda t, te: (te[t], 0, 0)),
            pl.BlockSpec((1, H, O2), lambda t, te: (te[t], 0, 0)),
            pl.BlockSpec((1, 1, O2), lambda t, te: (te[t], 0, 0)),
        ],
        out_specs=pl.BlockSpec((T, O2), lambda t, te: (t, 0)),
    )
    w2p = jnp.pad(ex_W2, ((0, 0), (0, 0), (0, O2 - O)))
    b2p = jnp.pad(ex_b2, ((0, 0), (0, O2 - O))).reshape(E, 1, O2)
    return pl.pallas_call(
        _expert_mlp_body,
        grid_spec=grid_spec,
        out_shape=jax.ShapeDtypeStruct((P, O2), _F32),
    )(tile_expert, x_sorted, ex_W1, ex_b1.reshape(E, 1, H), w2p, b2p)


def _heads_body(sh_ref, r3_ref, w_ref, qw_ref, qb_ref, lw_ref, lb_ref,
                quad_ref, ll_ref, comb_ref):
    j = pl.program_id(1)

    @pl.when(j == 0)
    def _():
        r3 = r3_ref[...]
        w = w_ref[...]
        routed = (r3[:, 0, :] * w[:, 0:1]
                  + r3[:, 1, :] * w[:, 1:2]
                  + r3[:, 2, :] * w[:, 2:3])
        comb = sh_ref[...] + routed[:, 0:O]
        comb_ref[...] = comb
        ll_ref[...] = jnp.dot(comb, lw_ref[...], preferred_element_type=_F32) + lb_ref[...]

    quad_ref[...] = (jnp.dot(comb_ref[...], qw_ref[...], preferred_element_type=_F32)
                     + qb_ref[...])


def _heads(shared, routed3, topk_w, qt_W, qt_b, ll_W, ll_b):
    grid = (B // TB2, pl.cdiv(Q, TQ))
    return pl.pallas_call(
        _heads_body,
        grid=grid,
        in_specs=[
            pl.BlockSpec((TB2, O), lambda i, j: (i, 0)),
            pl.BlockSpec((TB2, K, O2), lambda i, j: (i, 0, 0)),
            pl.BlockSpec((TB2, 4), lambda i, j: (i, 0)),
            pl.BlockSpec((O, TQ), lambda i, j: (0, j)),
            pl.BlockSpec((1, TQ), lambda i, j: (0, j)),
            pl.BlockSpec((O, 2), lambda i, j: (0, 0)),
            pl.BlockSpec((1, 2), lambda i, j: (0, 0)),
        ],
        out_specs=[
            pl.BlockSpec((TB2, TQ), lambda i, j: (i, j)),
            pl.BlockSpec((TB2, 2), lambda i, j: (i, 0)),
        ],
        out_shape=[
            jax.ShapeDtypeStruct((B, Q), _F32),
            jax.ShapeDtypeStruct((B, 2), _F32),
        ],
        scratch_shapes=[pltpu.VMEM((TB2, O), _F32)],
    )(shared, routed3, topk_w, qt_W, qt_b.reshape(1, Q), ll_W, ll_b.reshape(1, 2))


def _dispatch_meta(topk_idx):
    mask_be = (topk_idx[:, :, None] == jnp.arange(E)[None, None, :]).any(axis=1)
    mask_be = mask_be.astype(_I32)
    csum = jnp.cumsum(mask_be, axis=0)
    rank_be = csum - mask_be
    counts = csum[-1]
    tiles_e = (counts + T - 1) // T
    tile_start = jnp.concatenate([jnp.zeros((1,), _I32),
                                  jnp.cumsum(tiles_e).astype(_I32)])
    pstart = tile_start[:E] * T
    dest_be = pstart[None, :] + rank_be
    dest3 = jnp.take_along_axis(dest_be, topk_idx, axis=1).astype(_I32)
    slot_token = jnp.zeros((P,), _I32).at[dest3.reshape(-1)].set(
        jnp.repeat(jnp.arange(B, dtype=_I32), K), mode="drop")
    tile_expert = jnp.searchsorted(
        jnp.cumsum(tiles_e), jnp.arange(NT, dtype=_I32), side="right")
    tile_expert = jnp.minimum(tile_expert, E - 1).astype(_I32)
    return dest3, slot_token, tile_expert, counts


def kernel(x, router_W, router_b, sh_W1, sh_b1, sh_W2, sh_b2,
           ex_W1, ex_b1, ex_W2, ex_b2, ll_W, ll_b, qt_W, qt_b):
    topk_idx8, topk_w, shared, stats = _router_shared(
        x, router_W, router_b, sh_W1, sh_b1, sh_W2, sh_b2)
    topk_idx = topk_idx8[:, :K]
    dest3, slot_token, tile_expert, counts = _dispatch_meta(topk_idx)

    x_sorted = _make_sc_gather(P, D, CH)(x, slot_token)
    out_sorted = _expert_mlp(x_sorted, tile_expert, ex_W1, ex_b1, ex_W2, ex_b2)
    routed3 = _make_sc_gather(B * K, O2, BKW)(out_sorted,
                                              dest3.reshape(-1)).reshape(B, K, O2)

    quad, ll = _heads(shared, routed3, topk_w, qt_W, qt_b, ll_W, ll_b)

    p_mean = stats[0, 0:E] / B
    f_i = (E / (K * B)) * counts.astype(_F32)
    balance_loss = jnp.sum(f_i * p_mean) * 0.001
    return ll[:, 0], ll[:, 1], quad, balance_loss


# speedup vs baseline: 1.0087x; 1.0087x over previous
import functools

import jax
import jax.numpy as jnp
from jax import lax
from jax.experimental import pallas as pl
from jax.experimental.pallas import tpu as pltpu
from jax.experimental.pallas import tpu_sc as plsc

B = 8192
D = 1024
E = 32
K = 3
H = 256
O = 64
O2 = 128
Q = 11399

T = 256
NT = (B * K) // T + E
P = NT * T
TB1 = 512
TB2 = 512
TQ = 512

NW = 32
RPW = P // NW
CH = 32
NCH = RPW // CH
BKW = (B * K) // NW

_F32 = jnp.float32
_I32 = jnp.int32


def _gelu(x):
    return 0.5 * x * (1.0 + lax.erf(x * 0.7071067811865476))


def _router_shared_body(x_ref, rw_ref, rb_ref, w1_ref, b1_ref, w2_ref, b2_ref,
                        idx_ref, w_ref, sh_ref, stats_ref):
    i = pl.program_id(0)
    x = x_ref[...]
    logits = jnp.dot(x, rw_ref[...], preferred_element_type=_F32) + rb_ref[...]
    m = jnp.max(logits, axis=1, keepdims=True)
    ex = jnp.exp(logits - m)
    probs = ex / jnp.sum(ex, axis=1, keepdims=True)

    @pl.when(i == 0)
    def _():
        stats_ref[...] = jnp.zeros_like(stats_ref)

    stats_ref[0:1, 0:E] += jnp.sum(probs, axis=0, keepdims=True)

    iota = lax.broadcasted_iota(_I32, probs.shape, 1)
    pw = probs
    cols_i = []
    cols_w = []
    for _ in range(K):
        mk = jnp.max(pw, axis=1, keepdims=True)
        a = jnp.min(jnp.where(pw == mk, iota, E), axis=1, keepdims=True)
        cols_i.append(a)
        cols_w.append(mk)
        pw = jnp.where(iota == a, -1.0, pw)
    zi = jnp.zeros_like(cols_i[0])
    zw = jnp.zeros_like(cols_w[0])
    idx_full = jnp.concatenate(cols_i + [zi], axis=1)
    w_full = jnp.concatenate(cols_w + [zw], axis=1)
    wsum = cols_w[0] + cols_w[1] + cols_w[2]
    idx_ref[...] = idx_full
    w_ref[...] = w_full / wsum

    mid = _gelu(jnp.dot(x, w1_ref[...], preferred_element_type=_F32) + b1_ref[...])
    sh_ref[...] = jnp.dot(mid, w2_ref[...], preferred_element_type=_F32) + b2_ref[...]


def _router_shared(x, router_W, router_b, sh_W1, sh_b1, sh_W2, sh_b2):
    grid = (B // TB1,)
    return pl.pallas_call(
        _router_shared_body,
        grid=grid,
        in_specs=[
            pl.BlockSpec((TB1, D), lambda i: (i, 0)),
            pl.BlockSpec((D, E), lambda i: (0, 0)),
            pl.BlockSpec((1, E), lambda i: (0, 0)),
            pl.BlockSpec((D, H), lambda i: (0, 0)),
            pl.BlockSpec((1, H), lambda i: (0, 0)),
            pl.BlockSpec((H, O), lambda i: (0, 0)),
            pl.BlockSpec((1, O), lambda i: (0, 0)),
        ],
        out_specs=[
            pl.BlockSpec((TB1, 4), lambda i: (i, 0)),
            pl.BlockSpec((TB1, 4), lambda i: (i, 0)),
            pl.BlockSpec((TB1, O), lambda i: (i, 0)),
            pl.BlockSpec((8, 128), lambda i: (0, 0)),
        ],
        out_shape=[
            jax.ShapeDtypeStruct((B, 4), _I32),
            jax.ShapeDtypeStruct((B, 4), _F32),
            jax.ShapeDtypeStruct((B, O), _F32),
            jax.ShapeDtypeStruct((8, 128), _F32),
        ],
    )(x, router_W, router_b.reshape(1, E), sh_W1, sh_b1.reshape(1, H),
      sh_W2, sh_b2.reshape(1, O))


@functools.cache
def _make_sc_gather(n_rows, width, chunk, nbuf=3):
    rpw = n_rows // NW
    nch = rpw // chunk
    mesh = plsc.VectorSubcoreMesh(core_axis_name="c", subcore_axis_name="s")
    buf_types = [pltpu.VMEM((chunk, width), _F32) for _ in range(nbuf)]
    gsem_types = [pltpu.SemaphoreType.DMA for _ in range(nbuf)]
    ssem_types = [pltpu.SemaphoreType.DMA for _ in range(nbuf)]

    @functools.partial(
        pl.kernel,
        mesh=mesh,
        out_type=jax.ShapeDtypeStruct((n_rows, width), _F32),
        scratch_types=[pltpu.VMEM((rpw,), _I32)] + buf_types + gsem_types + ssem_types,
    )
    def sc_gather(src_hbm, idx_hbm, out_hbm, idx_v, *rest):
        bufs = rest[:nbuf]
        gsems = rest[nbuf:2 * nbuf]
        ssems = rest[2 * nbuf:]
        wid = lax.axis_index("s") * 2 + lax.axis_index("c")
        base = wid * rpw
        pltpu.sync_copy(idx_hbm.at[pl.ds(base, rpw)], idx_v)
        g_h = [None] * nch
        s_h = [None] * nch
        for c in range(nch):
            b = c % nbuf
            if c >= nbuf:
                s_h[c - nbuf].wait()
            g_h[c] = pltpu.async_copy(
                src_hbm.at[idx_v.at[pl.ds(c * chunk, chunk)]], bufs[b], gsems[b])
            if c >= 1:
                pb = (c - 1) % nbuf
                g_h[c - 1].wait()
                s_h[c - 1] = pltpu.async_copy(
                    bufs[pb], out_hbm.at[pl.ds(base + (c - 1) * chunk, chunk)],
                    ssems[pb])
        lb = (nch - 1) % nbuf
        g_h[nch - 1].wait()
        s_h[nch - 1] = pltpu.async_copy(
            bufs[lb], out_hbm.at[pl.ds(base + (nch - 1) * chunk, chunk)], ssems[lb])
        for c in range(max(0, nch - nbuf), nch):
            s_h[c].wait()

    return sc_gather


def _expert_mlp_body(te_ref, xs_ref, w1_ref, b1_ref, w2_ref, b2_ref, out_ref):
    h = _gelu(jnp.dot(xs_ref[...], w1_ref[0], preferred_element_type=_F32)
              + b1_ref[0])
    out_ref[...] = jnp.dot(h, w2_ref[0], preferred_element_type=_F32) + b2_ref[0]


def _expert_mlp(x_sorted, tile_expert, ex_W1, ex_b1, ex_W2, ex_b2):
    grid_spec = pltpu.PrefetchScalarGridSpec(
        num_scalar_prefetch=1,
        grid=(NT,),
        in_specs=[
            pl.BlockSpec((T, D), lambda t, te: (t, 0)),
            pl.BlockSpec((1, D, H), lambda t, te: (te[t], 0, 0)),
            pl.BlockSpec((1, 1, H), lambda t, te: (te[t], 0, 0)),
            pl.BlockSpec((1, H, O2), lambda t, te: (te[t], 0, 0)),
            pl.BlockSpec((1, 1, O2), lambda t, te: (te[t], 0, 0)),
        ],
        out_specs=pl.BlockSpec((T, O2), lambda t, te: (t, 0)),
    )
    w2p = jnp.pad(ex_W2, ((0, 0), (0, 0), (0, O2 - O)))
    b2p = jnp.pad(ex_b2, ((0, 0), (0, O2 - O))).reshape(E, 1, O2)
    return pl.pallas_call(
        _expert_mlp_body,
        grid_spec=grid_spec,
        out_shape=jax.ShapeDtypeStruct((P, O2), _F32),
    )(tile_expert, x_sorted, ex_W1, ex_b1.reshape(E, 1, H), w2p, b2p)


def _heads_body(sh_ref, r3_ref, w_ref, qw_ref, qb_ref, lw_ref, lb_ref,
                quad_ref, ll_ref, comb_ref):
    j = pl.program_id(1)

    @pl.when(j == 0)
    def _():
        r3 = r3_ref[...]
        w = w_ref[...]
        routed = (r3[:, 0, :] * w[:, 0:1]
                  + r3[:, 1, :] * w[:, 1:2]
                  + r3[:, 2, :] * w[:, 2:3])
        comb = sh_ref[...] + routed[:, 0:O]
        comb_ref[...] = comb
        ll_ref[...] = jnp.dot(comb, lw_ref[...], preferred_element_type=_F32) + lb_ref[...]

    quad_ref[...] = (jnp.dot(comb_ref[...], qw_ref[...], preferred_element_type=_F32)
                     + qb_ref[...])


def _heads(shared, routed3, topk_w, qt_W, qt_b, ll_W, ll_b):
    grid = (B // TB2, pl.cdiv(Q, TQ))
    return pl.pallas_call(
        _heads_body,
        grid=grid,
        in_specs=[
            pl.BlockSpec((TB2, O), lambda i, j: (i, 0)),
            pl.BlockSpec((TB2, K, O2), lambda i, j: (i, 0, 0)),
            pl.BlockSpec((TB2, 4), lambda i, j: (i, 0)),
            pl.BlockSpec((O, TQ), lambda i, j: (0, j)),
            pl.BlockSpec((1, TQ), lambda i, j: (0, j)),
            pl.BlockSpec((O, 2), lambda i, j: (0, 0)),
            pl.BlockSpec((1, 2), lambda i, j: (0, 0)),
        ],
        out_specs=[
            pl.BlockSpec((TB2, TQ), lambda i, j: (i, j)),
            pl.BlockSpec((TB2, 2), lambda i, j: (i, 0)),
        ],
        out_shape=[
            jax.ShapeDtypeStruct((B, Q), _F32),
            jax.ShapeDtypeStruct((B, 2), _F32),
        ],
        scratch_shapes=[pltpu.VMEM((TB2, O), _F32)],
    )(shared, routed3, topk_w, qt_W, qt_b.reshape(1, Q), ll_W, ll_b.reshape(1, 2))


def _dispatch_meta(topk_idx):
    mask_be = (topk_idx[:, :, None] == jnp.arange(E)[None, None, :]).any(axis=1)
    mask_be = mask_be.astype(_I32)
    csum = jnp.cumsum(mask_be, axis=0)
    rank_be = csum - mask_be
    counts = csum[-1]
    tiles_e = (counts + T - 1) // T
    tile_start = jnp.concatenate([jnp.zeros((1,), _I32),
                                  jnp.cumsum(tiles_e).astype(_I32)])
    pstart = tile_start[:E] * T
    dest_be = pstart[None, :] + rank_be
    dest3 = jnp.take_along_axis(dest_be, topk_idx, axis=1).astype(_I32)
    slot_token = jnp.zeros((P,), _I32).at[dest3.reshape(-1)].set(
        jnp.repeat(jnp.arange(B, dtype=_I32), K), mode="drop")
    tile_expert = jnp.searchsorted(
        jnp.cumsum(tiles_e), jnp.arange(NT, dtype=_I32), side="right")
    tile_expert = jnp.minimum(tile_expert, E - 1).astype(_I32)
    return dest3, slot_token, tile_expert, counts


def kernel(x, router_W, router_b, sh_W1, sh_b1, sh_W2, sh_b2,
           ex_W1, ex_b1, ex_W2, ex_b2, ll_W, ll_b, qt_W, qt_b):
    topk_idx8, topk_w, shared, stats = _router_shared(
        x, router_W, router_b, sh_W1, sh_b1, sh_W2, sh_b2)
    topk_idx = topk_idx8[:, :K]
    dest3, slot_token, tile_expert, counts = _dispatch_meta(topk_idx)

    x_sorted = _make_sc_gather(P, D, CH)(x, slot_token)
    out_sorted = _expert_mlp(x_sorted, tile_expert, ex_W1, ex_b1, ex_W2, ex_b2)
    routed3 = _make_sc_gather(B * K, O2, BKW, nbuf=1)(
        out_sorted, dest3.reshape(-1)).reshape(B, K, O2)

    quad, ll = _heads(shared, routed3, topk_w, qt_W, qt_b, ll_W, ll_b)

    p_mean = stats[0, 0:E] / B
    f_i = (E / (K * B)) * counts.astype(_F32)
    balance_loss = jnp.sum(f_i * p_mean) * 0.001
    return ll[:, 0], ll[:, 1], quad, balance_loss


# speedup vs baseline: 1.0782x; 1.0689x over previous
import functools

import jax
import jax.numpy as jnp
from jax import lax
from jax.experimental import pallas as pl
from jax.experimental.pallas import tpu as pltpu
from jax.experimental.pallas import tpu_sc as plsc

B = 8192
D = 1024
E = 32
K = 3
H = 256
O = 64
O2 = 128
Q = 11399

T = 256
NT = (B * K) // T + E
P = NT * T
TB1 = 512
TB2 = 512
TQ = 512

NW = 32
RPW = P // NW
CH = 32
NCH = RPW // CH
BKW = (B * K) // NW

_F32 = jnp.float32
_I32 = jnp.int32


def _gelu(x):
    return 0.5 * x * (1.0 + lax.erf(x * 0.7071067811865476))


def _router_shared_body(x_ref, rw_ref, rb_ref, w1_ref, b1_ref, w2_ref, b2_ref,
                        idx_ref, w_ref, sh_ref, stats_ref, xb_ref):
    i = pl.program_id(0)
    x = x_ref[...]
    u1 = lax.bitcast_convert_type(x[:, :D // 2], jnp.uint32) + jnp.uint32(0x8000)
    u2 = lax.bitcast_convert_type(x[:, D // 2:], jnp.uint32) + jnp.uint32(0x8000)
    packed = (u1 & jnp.uint32(0xFFFF0000)) | (u2 >> 16)
    xb_ref[...] = lax.bitcast_convert_type(packed, _F32)
    logits = jnp.dot(x, rw_ref[...], preferred_element_type=_F32) + rb_ref[...]
    m = jnp.max(logits, axis=1, keepdims=True)
    ex = jnp.exp(logits - m)
    probs = ex / jnp.sum(ex, axis=1, keepdims=True)

    @pl.when(i == 0)
    def _():
        stats_ref[...] = jnp.zeros_like(stats_ref)

    stats_ref[0:1, 0:E] += jnp.sum(probs, axis=0, keepdims=True)

    iota = lax.broadcasted_iota(_I32, probs.shape, 1)
    pw = probs
    cols_i = []
    cols_w = []
    for _ in range(K):
        mk = jnp.max(pw, axis=1, keepdims=True)
        a = jnp.min(jnp.where(pw == mk, iota, E), axis=1, keepdims=True)
        cols_i.append(a)
        cols_w.append(mk)
        pw = jnp.where(iota == a, -1.0, pw)
    zi = jnp.zeros_like(cols_i[0])
    zw = jnp.zeros_like(cols_w[0])
    idx_full = jnp.concatenate(cols_i + [zi], axis=1)
    w_full = jnp.concatenate(cols_w + [zw], axis=1)
    wsum = cols_w[0] + cols_w[1] + cols_w[2]
    idx_ref[...] = idx_full
    w_ref[...] = w_full / wsum

    mid = _gelu(jnp.dot(x, w1_ref[...], preferred_element_type=_F32) + b1_ref[...])
    sh_ref[...] = jnp.dot(mid, w2_ref[...], preferred_element_type=_F32) + b2_ref[...]


def _router_shared(x, router_W, router_b, sh_W1, sh_b1, sh_W2, sh_b2):
    grid = (B // TB1,)
    return pl.pallas_call(
        _router_shared_body,
        grid=grid,
        in_specs=[
            pl.BlockSpec((TB1, D), lambda i: (i, 0)),
            pl.BlockSpec((D, E), lambda i: (0, 0)),
            pl.BlockSpec((1, E), lambda i: (0, 0)),
            pl.BlockSpec((D, H), lambda i: (0, 0)),
            pl.BlockSpec((1, H), lambda i: (0, 0)),
            pl.BlockSpec((H, O), lambda i: (0, 0)),
            pl.BlockSpec((1, O), lambda i: (0, 0)),
        ],
        out_specs=[
            pl.BlockSpec((TB1, 4), lambda i: (i, 0)),
            pl.BlockSpec((TB1, 4), lambda i: (i, 0)),
            pl.BlockSpec((TB1, O), lambda i: (i, 0)),
            pl.BlockSpec((8, 128), lambda i: (0, 0)),
            pl.BlockSpec((TB1, D // 2), lambda i: (i, 0)),
        ],
        out_shape=[
            jax.ShapeDtypeStruct((B, 4), _I32),
            jax.ShapeDtypeStruct((B, 4), _F32),
            jax.ShapeDtypeStruct((B, O), _F32),
            jax.ShapeDtypeStruct((8, 128), _F32),
            jax.ShapeDtypeStruct((B, D // 2), _F32),
        ],
    )(x, router_W, router_b.reshape(1, E), sh_W1, sh_b1.reshape(1, H),
      sh_W2, sh_b2.reshape(1, O))


@functools.cache
def _make_sc_gather(n_rows, width, chunk, nbuf=4, dtype=_F32):
    rpw = n_rows // NW
    nch = rpw // chunk
    mesh = plsc.VectorSubcoreMesh(core_axis_name="c", subcore_axis_name="s")
    buf_types = [pltpu.VMEM((chunk, width), dtype) for _ in range(nbuf)]
    gsem_types = [pltpu.SemaphoreType.DMA for _ in range(nbuf)]
    ssem_types = [pltpu.SemaphoreType.DMA for _ in range(nbuf)]

    @functools.partial(
        pl.kernel,
        mesh=mesh,
        out_type=jax.ShapeDtypeStruct((n_rows, width), dtype),
        scratch_types=[pltpu.VMEM((rpw,), _I32)] + buf_types + gsem_types + ssem_types,
    )
    def sc_gather(src_hbm, idx_hbm, out_hbm, idx_v, *rest):
        bufs = rest[:nbuf]
        gsems = rest[nbuf:2 * nbuf]
        ssems = rest[2 * nbuf:]
        wid = lax.axis_index("s") * 2 + lax.axis_index("c")
        base = wid * rpw
        pltpu.sync_copy(idx_hbm.at[pl.ds(base, rpw)], idx_v)

        def gather(c):
            return pltpu.async_copy(
                src_hbm.at[idx_v.at[pl.ds(c * chunk, chunk)]],
                bufs[c % nbuf], gsems[c % nbuf])

        g_h = [None] * nch
        s_h = [None] * nch
        for c in range(min(nbuf, nch)):
            g_h[c] = gather(c)
        for c in range(nch):
            g_h[c].wait()
            s_h[c] = pltpu.async_copy(
                bufs[c % nbuf], out_hbm.at[pl.ds(base + c * chunk, chunk)],
                ssems[c % nbuf])
            if c + nbuf < nch:
                s_h[c].wait()
                g_h[c + nbuf] = gather(c + nbuf)
        for c in range(max(0, nch - nbuf), nch):
            s_h[c].wait()

    return sc_gather


def _expert_mlp_body(te_ref, xs_ref, w1_ref, b1_ref, w2_ref, b2_ref, out_ref):
    v = lax.bitcast_convert_type(xs_ref[...], jnp.uint32)
    xa = lax.bitcast_convert_type(v & jnp.uint32(0xFFFF0000), _F32)
    xb = lax.bitcast_convert_type(v << 16, _F32)
    xs = jnp.concatenate([xa, xb], axis=1)
    h = _gelu(jnp.dot(xs, w1_ref[0], preferred_element_type=_F32)
              + b1_ref[0])
    out_ref[...] = jnp.dot(h, w2_ref[0], preferred_element_type=_F32) + b2_ref[0]


def _expert_mlp(x_sorted, tile_expert, ex_W1, ex_b1, ex_W2, ex_b2):
    grid_spec = pltpu.PrefetchScalarGridSpec(
        num_scalar_prefetch=1,
        grid=(NT,),
        in_specs=[
            pl.BlockSpec((T, D // 2), lambda t, te: (t, 0)),
            pl.BlockSpec((1, D, H), lambda t, te: (te[t], 0, 0)),
            pl.BlockSpec((1, 1, H), lambda t, te: (te[t], 0, 0)),
            pl.BlockSpec((1, H, O2), lambda t, te: (te[t], 0, 0)),
            pl.BlockSpec((1, 1, O2), lambda t, te: (te[t], 0, 0)),
        ],
        out_specs=pl.BlockSpec((T, O2), lambda t, te: (t, 0)),
    )
    w2p = jnp.pad(ex_W2, ((0, 0), (0, 0), (0, O2 - O)))
    b2p = jnp.pad(ex_b2, ((0, 0), (0, O2 - O))).reshape(E, 1, O2)
    return pl.pallas_call(
        _expert_mlp_body,
        grid_spec=grid_spec,
        out_shape=jax.ShapeDtypeStruct((P, O2), _F32),
    )(tile_expert, x_sorted, ex_W1, ex_b1.reshape(E, 1, H), w2p, b2p)


def _heads_body(sh_ref, r3_ref, w_ref, qw_ref, qb_ref, lw_ref, lb_ref,
                quad_ref, ll_ref, comb_ref):
    j = pl.program_id(1)

    @pl.when(j == 0)
    def _():
        r3 = r3_ref[...]
        w = w_ref[...]
        routed = (r3[:, 0, :] * w[:, 0:1]
                  + r3[:, 1, :] * w[:, 1:2]
                  + r3[:, 2, :] * w[:, 2:3])
        comb = sh_ref[...] + routed[:, 0:O]
        comb_ref[...] = comb
        ll_ref[...] = jnp.dot(comb, lw_ref[...], preferred_element_type=_F32) + lb_ref[...]

    quad_ref[...] = (jnp.dot(comb_ref[...], qw_ref[...], preferred_element_type=_F32)
                     + qb_ref[...])


def _heads(shared, routed3, topk_w, qt_W, qt_b, ll_W, ll_b):
    grid = (B // TB2, pl.cdiv(Q, TQ))
    return pl.pallas_call(
        _heads_body,
        grid=grid,
        in_specs=[
            pl.BlockSpec((TB2, O), lambda i, j: (i, 0)),
            pl.BlockSpec((TB2, K, O2), lambda i, j: (i, 0, 0)),
            pl.BlockSpec((TB2, 4), lambda i, j: (i, 0)),
            pl.BlockSpec((O, TQ), lambda i, j: (0, j)),
            pl.BlockSpec((1, TQ), lambda i, j: (0, j)),
            pl.BlockSpec((O, 2), lambda i, j: (0, 0)),
            pl.BlockSpec((1, 2), lambda i, j: (0, 0)),
        ],
        out_specs=[
            pl.BlockSpec((TB2, TQ), lambda i, j: (i, j)),
            pl.BlockSpec((TB2, 2), lambda i, j: (i, 0)),
        ],
        out_shape=[
            jax.ShapeDtypeStruct((B, Q), _F32),
            jax.ShapeDtypeStruct((B, 2), _F32),
        ],
        scratch_shapes=[pltpu.VMEM((TB2, O), _F32)],
    )(shared, routed3, topk_w, qt_W, qt_b.reshape(1, Q), ll_W, ll_b.reshape(1, 2))


def _dispatch_meta(topk_idx):
    mask_be = (topk_idx[:, :, None] == jnp.arange(E)[None, None, :]).any(axis=1)
    mask_be = mask_be.astype(_I32)
    csum = jnp.cumsum(mask_be, axis=0)
    rank_be = csum - mask_be
    counts = csum[-1]
    tiles_e = (counts + T - 1) // T
    tile_start = jnp.concatenate([jnp.zeros((1,), _I32),
                                  jnp.cumsum(tiles_e).astype(_I32)])
    pstart = tile_start[:E] * T
    dest_be = pstart[None, :] + rank_be
    dest3 = jnp.take_along_axis(dest_be, topk_idx, axis=1).astype(_I32)
    slot_token = jnp.zeros((P,), _I32).at[dest3.reshape(-1)].set(
        jnp.repeat(jnp.arange(B, dtype=_I32), K), mode="drop")
    tile_expert = jnp.searchsorted(
        jnp.cumsum(tiles_e), jnp.arange(NT, dtype=_I32), side="right")
    tile_expert = jnp.minimum(tile_expert, E - 1).astype(_I32)
    return dest3, slot_token, tile_expert, counts


def kernel(x, router_W, router_b, sh_W1, sh_b1, sh_W2, sh_b2,
           ex_W1, ex_b1, ex_W2, ex_b2, ll_W, ll_b, qt_W, qt_b):
    topk_idx8, topk_w, shared, stats, x_bf = _router_shared(
        x, router_W, router_b, sh_W1, sh_b1, sh_W2, sh_b2)
    topk_idx = topk_idx8[:, :K]
    dest3, slot_token, tile_expert, counts = _dispatch_meta(topk_idx)

    x_sorted = _make_sc_gather(P, D // 2, CH)(x_bf, slot_token)
    out_sorted = _expert_mlp(x_sorted, tile_expert, ex_W1, ex_b1, ex_W2, ex_b2)
    routed3 = _make_sc_gather(B * K, O2, BKW, nbuf=1)(
        out_sorted, dest3.reshape(-1)).reshape(B, K, O2)

    quad, ll = _heads(shared, routed3, topk_w, qt_W, qt_b, ll_W, ll_b)

    p_mean = stats[0, 0:E] / B
    f_i = (E / (K * B)) * counts.astype(_F32)
    balance_loss = jnp.sum(f_i * p_mean) * 0.001
    return ll[:, 0], ll[:, 1], quad, balance_loss


# speedup vs baseline: 1.1764x; 1.0911x over previous
import functools

import jax
import jax.numpy as jnp
from jax import lax
from jax.experimental import pallas as pl
from jax.experimental.pallas import tpu as pltpu
from jax.experimental.pallas import tpu_sc as plsc

B = 8192
D = 1024
E = 32
K = 3
H = 256
O = 64
O2 = 128
Q = 11399

T = 128
NT = (B * K) // T + E
P = NT * T
TB1 = 512
TB2 = 512
TQ = 512

NW = 32
RPW = P // NW
CH = 32
NCH = RPW // CH
BKW = (B * K) // NW

_F32 = jnp.float32
_I32 = jnp.int32


def _gelu(x):
    return 0.5 * x * (1.0 + lax.erf(x * 0.7071067811865476))


def _router_shared_body(x_ref, rw_ref, rb_ref, w1_ref, b1_ref, w2_ref, b2_ref,
                        idx_ref, w_ref, sh_ref, stats_ref, xb_ref):
    i = pl.program_id(0)
    x = x_ref[...]
    u1 = lax.bitcast_convert_type(x[:, :D // 2], jnp.uint32) + jnp.uint32(0x8000)
    u2 = lax.bitcast_convert_type(x[:, D // 2:], jnp.uint32) + jnp.uint32(0x8000)
    packed = (u1 & jnp.uint32(0xFFFF0000)) | (u2 >> 16)
    xb_ref[...] = lax.bitcast_convert_type(packed, _F32)
    logits = jnp.dot(x, rw_ref[...], preferred_element_type=_F32) + rb_ref[...]
    m = jnp.max(logits, axis=1, keepdims=True)
    ex = jnp.exp(logits - m)
    probs = ex / jnp.sum(ex, axis=1, keepdims=True)

    @pl.when(i == 0)
    def _():
        stats_ref[...] = jnp.zeros_like(stats_ref)

    stats_ref[0:1, 0:E] += jnp.sum(probs, axis=0, keepdims=True)

    iota = lax.broadcasted_iota(_I32, probs.shape, 1)
    pw = probs
    cols_i = []
    cols_w = []
    for _ in range(K):
        mk = jnp.max(pw, axis=1, keepdims=True)
        a = jnp.min(jnp.where(pw == mk, iota, E), axis=1, keepdims=True)
        cols_i.append(a)
        cols_w.append(mk)
        pw = jnp.where(iota == a, -1.0, pw)
    zi = jnp.zeros_like(cols_i[0])
    zw = jnp.zeros_like(cols_w[0])
    idx_full = jnp.concatenate(cols_i + [zi], axis=1)
    w_full = jnp.concatenate(cols_w + [zw], axis=1)
    wsum = cols_w[0] + cols_w[1] + cols_w[2]
    idx_ref[...] = idx_full
    w_ref[...] = w_full / wsum

    mid = _gelu(jnp.dot(x, w1_ref[...], preferred_element_type=_F32) + b1_ref[...])
    sh_ref[...] = jnp.dot(mid, w2_ref[...], preferred_element_type=_F32) + b2_ref[...]


def _router_shared(x, router_W, router_b, sh_W1, sh_b1, sh_W2, sh_b2):
    grid = (B // TB1,)
    return pl.pallas_call(
        _router_shared_body,
        grid=grid,
        in_specs=[
            pl.BlockSpec((TB1, D), lambda i: (i, 0)),
            pl.BlockSpec((D, E), lambda i: (0, 0)),
            pl.BlockSpec((1, E), lambda i: (0, 0)),
            pl.BlockSpec((D, H), lambda i: (0, 0)),
            pl.BlockSpec((1, H), lambda i: (0, 0)),
            pl.BlockSpec((H, O), lambda i: (0, 0)),
            pl.BlockSpec((1, O), lambda i: (0, 0)),
        ],
        out_specs=[
            pl.BlockSpec((TB1, 4), lambda i: (i, 0)),
            pl.BlockSpec((TB1, 4), lambda i: (i, 0)),
            pl.BlockSpec((TB1, O), lambda i: (i, 0)),
            pl.BlockSpec((8, 128), lambda i: (0, 0)),
            pl.BlockSpec((TB1, D // 2), lambda i: (i, 0)),
        ],
        out_shape=[
            jax.ShapeDtypeStruct((B, 4), _I32),
            jax.ShapeDtypeStruct((B, 4), _F32),
            jax.ShapeDtypeStruct((B, O), _F32),
            jax.ShapeDtypeStruct((8, 128), _F32),
            jax.ShapeDtypeStruct((B, D // 2), _F32),
        ],
    )(x, router_W, router_b.reshape(1, E), sh_W1, sh_b1.reshape(1, H),
      sh_W2, sh_b2.reshape(1, O))


@functools.cache
def _make_sc_gather(n_rows, width, chunk, nbuf=4, dtype=_F32):
    rpw = n_rows // NW
    nch = rpw // chunk
    mesh = plsc.VectorSubcoreMesh(core_axis_name="c", subcore_axis_name="s")
    buf_types = [pltpu.VMEM((chunk, width), dtype) for _ in range(nbuf)]
    gsem_types = [pltpu.SemaphoreType.DMA for _ in range(nbuf)]
    ssem_types = [pltpu.SemaphoreType.DMA for _ in range(nbuf)]

    @functools.partial(
        pl.kernel,
        mesh=mesh,
        out_type=jax.ShapeDtypeStruct((n_rows, width), dtype),
        scratch_types=[pltpu.VMEM((rpw,), _I32)] + buf_types + gsem_types + ssem_types,
    )
    def sc_gather(src_hbm, idx_hbm, out_hbm, idx_v, *rest):
        bufs = rest[:nbuf]
        gsems = rest[nbuf:2 * nbuf]
        ssems = rest[2 * nbuf:]
        wid = lax.axis_index("s") * 2 + lax.axis_index("c")
        base = wid * rpw
        pltpu.sync_copy(idx_hbm.at[pl.ds(base, rpw)], idx_v)

        def gather(c):
            return pltpu.async_copy(
                src_hbm.at[idx_v.at[pl.ds(c * chunk, chunk)]],
                bufs[c % nbuf], gsems[c % nbuf])

        g_h = [None] * nch
        s_h = [None] * nch
        for c in range(min(nbuf, nch)):
            g_h[c] = gather(c)
        for c in range(nch):
            g_h[c].wait()
            s_h[c] = pltpu.async_copy(
                bufs[c % nbuf], out_hbm.at[pl.ds(base + c * chunk, chunk)],
                ssems[c % nbuf])
            if c + nbuf < nch:
                s_h[c].wait()
                g_h[c + nbuf] = gather(c + nbuf)
        for c in range(max(0, nch - nbuf), nch):
            s_h[c].wait()

    return sc_gather


def _expert_mlp_body(te_ref, xs_ref, w1_ref, b1_ref, w2_ref, b2_ref, out_ref):
    v = lax.bitcast_convert_type(xs_ref[...], jnp.uint32)
    xa = lax.bitcast_convert_type(v & jnp.uint32(0xFFFF0000), _F32)
    xb = lax.bitcast_convert_type(v << 16, _F32)
    xs = jnp.concatenate([xa, xb], axis=1)
    h = _gelu(jnp.dot(xs, w1_ref[0], preferred_element_type=_F32)
              + b1_ref[0])
    out_ref[...] = jnp.dot(h, w2_ref[0], preferred_element_type=_F32) + b2_ref[0]


def _expert_mlp(x_sorted, tile_expert, ex_W1, ex_b1, ex_W2, ex_b2):
    grid_spec = pltpu.PrefetchScalarGridSpec(
        num_scalar_prefetch=1,
        grid=(NT,),
        in_specs=[
            pl.BlockSpec((T, D // 2), lambda t, te: (t, 0)),
            pl.BlockSpec((1, D, H), lambda t, te: (te[t], 0, 0)),
            pl.BlockSpec((1, 1, H), lambda t, te: (te[t], 0, 0)),
            pl.BlockSpec((1, H, O2), lambda t, te: (te[t], 0, 0)),
            pl.BlockSpec((1, 1, O2), lambda t, te: (te[t], 0, 0)),
        ],
        out_specs=pl.BlockSpec((T, O2), lambda t, te: (t, 0)),
    )
    w2p = jnp.pad(ex_W2, ((0, 0), (0, 0), (0, O2 - O)))
    b2p = jnp.pad(ex_b2, ((0, 0), (0, O2 - O))).reshape(E, 1, O2)
    return pl.pallas_call(
        _expert_mlp_body,
        grid_spec=grid_spec,
        out_shape=jax.ShapeDtypeStruct((P, O2), _F32),
    )(tile_expert, x_sorted, ex_W1, ex_b1.reshape(E, 1, H), w2p, b2p)


def _heads_body(sh_ref, r3_ref, w_ref, qw_ref, qb_ref, lw_ref, lb_ref,
                quad_ref, ll_ref, comb_ref):
    j = pl.program_id(1)

    @pl.when(j == 0)
    def _():
        r3 = r3_ref[...]
        w = w_ref[...]
        routed = (r3[:, 0, :] * w[:, 0:1]
                  + r3[:, 1, :] * w[:, 1:2]
                  + r3[:, 2, :] * w[:, 2:3])
        comb = sh_ref[...] + routed[:, 0:O]
        comb_ref[...] = comb
        ll_ref[...] = jnp.dot(comb, lw_ref[...], preferred_element_type=_F32) + lb_ref[...]

    quad_ref[...] = (jnp.dot(comb_ref[...], qw_ref[...], preferred_element_type=_F32)
                     + qb_ref[...])


def _heads(shared, routed3, topk_w, qt_W, qt_b, ll_W, ll_b):
    grid = (B // TB2, pl.cdiv(Q, TQ))
    return pl.pallas_call(
        _heads_body,
        grid=grid,
        in_specs=[
            pl.BlockSpec((TB2, O), lambda i, j: (i, 0)),
            pl.BlockSpec((TB2, K, O2), lambda i, j: (i, 0, 0)),
            pl.BlockSpec((TB2, 4), lambda i, j: (i, 0)),
            pl.BlockSpec((O, TQ), lambda i, j: (0, j)),
            pl.BlockSpec((1, TQ), lambda i, j: (0, j)),
            pl.BlockSpec((O, 2), lambda i, j: (0, 0)),
            pl.BlockSpec((1, 2), lambda i, j: (0, 0)),
        ],
        out_specs=[
            pl.BlockSpec((TB2, TQ), lambda i, j: (i, j)),
            pl.BlockSpec((TB2, 2), lambda i, j: (i, 0)),
        ],
        out_shape=[
            jax.ShapeDtypeStruct((B, Q), _F32),
            jax.ShapeDtypeStruct((B, 2), _F32),
        ],
        scratch_shapes=[pltpu.VMEM((TB2, O), _F32)],
    )(shared, routed3, topk_w, qt_W, qt_b.reshape(1, Q), ll_W, ll_b.reshape(1, 2))


def _dispatch_meta(topk_idx):
    mask_be = (topk_idx[:, :, None] == jnp.arange(E)[None, None, :]).any(axis=1)
    mask_be = mask_be.astype(_I32)
    csum = jnp.cumsum(mask_be, axis=0)
    rank_be = csum - mask_be
    counts = csum[-1]
    tiles_e = (counts + T - 1) // T
    tile_start = jnp.concatenate([jnp.zeros((1,), _I32),
                                  jnp.cumsum(tiles_e).astype(_I32)])
    pstart = tile_start[:E] * T
    dest_be = pstart[None, :] + rank_be
    dest3 = jnp.take_along_axis(dest_be, topk_idx, axis=1).astype(_I32)
    slot_token = jnp.zeros((P,), _I32).at[dest3.reshape(-1)].set(
        jnp.repeat(jnp.arange(B, dtype=_I32), K), mode="drop")
    tile_expert = jnp.searchsorted(
        jnp.cumsum(tiles_e), jnp.arange(NT, dtype=_I32), side="right")
    tile_expert = jnp.minimum(tile_expert, E - 1).astype(_I32)
    return dest3, slot_token, tile_expert, counts


def kernel(x, router_W, router_b, sh_W1, sh_b1, sh_W2, sh_b2,
           ex_W1, ex_b1, ex_W2, ex_b2, ll_W, ll_b, qt_W, qt_b):
    topk_idx8, topk_w, shared, stats, x_bf = _router_shared(
        x, router_W, router_b, sh_W1, sh_b1, sh_W2, sh_b2)
    topk_idx = topk_idx8[:, :K]
    dest3, slot_token, tile_expert, counts = _dispatch_meta(topk_idx)

    x_sorted = _make_sc_gather(P, D // 2, CH)(x_bf, slot_token)
    out_sorted = _expert_mlp(x_sorted, tile_expert, ex_W1, ex_b1, ex_W2, ex_b2)
    routed3 = _make_sc_gather(B * K, O2, BKW, nbuf=1)(
        out_sorted, dest3.reshape(-1)).reshape(B, K, O2)

    quad, ll = _heads(shared, routed3, topk_w, qt_W, qt_b, ll_W, ll_b)

    p_mean = stats[0, 0:E] / B
    f_i = (E / (K * B)) * counts.astype(_F32)
    balance_loss = jnp.sum(f_i * p_mean) * 0.001
    return ll[:, 0], ll[:, 1], quad, balance_loss


# speedup vs baseline: 1.3356x; 1.1353x over previous
import functools

import jax
import jax.numpy as jnp
from jax import lax
from jax.experimental import pallas as pl
from jax.experimental.pallas import tpu as pltpu
from jax.experimental.pallas import tpu_sc as plsc

B = 8192
D = 1024
E = 32
K = 3
H = 256
O = 64
O2 = 128
Q = 11399

T = 128
NT = (B * K) // T + E
P = NT * T
TB1 = 512
TB2 = 1024
TQ = 1024

NW = 32
RPW = P // NW
CH = 32
NCH = RPW // CH
BKW = (B * K) // NW

_F32 = jnp.float32
_I32 = jnp.int32


def _gelu(x):
    return 0.5 * x * (1.0 + lax.erf(x * 0.7071067811865476))


def _router_shared_body(x_ref, rw_ref, rb_ref, w1_ref, b1_ref, w2_ref, b2_ref,
                        idx_ref, w_ref, sh_ref, stats_ref, xb_ref):
    i = pl.program_id(0)
    x = x_ref[...]
    u1 = lax.bitcast_convert_type(x[:, :D // 2], jnp.uint32) + jnp.uint32(0x8000)
    u2 = lax.bitcast_convert_type(x[:, D // 2:], jnp.uint32) + jnp.uint32(0x8000)
    packed = (u1 & jnp.uint32(0xFFFF0000)) | (u2 >> 16)
    xb_ref[...] = lax.bitcast_convert_type(packed, _F32)
    logits = jnp.dot(x, rw_ref[...], preferred_element_type=_F32) + rb_ref[...]
    m = jnp.max(logits, axis=1, keepdims=True)
    ex = jnp.exp(logits - m)
    probs = ex / jnp.sum(ex, axis=1, keepdims=True)

    @pl.when(i == 0)
    def _():
        stats_ref[...] = jnp.zeros_like(stats_ref)

    stats_ref[0:1, 0:E] += jnp.sum(probs, axis=0, keepdims=True)

    iota = lax.broadcasted_iota(_I32, probs.shape, 1)
    pw = probs
    cols_i = []
    cols_w = []
    for _ in range(K):
        mk = jnp.max(pw, axis=1, keepdims=True)
        a = jnp.min(jnp.where(pw == mk, iota, E), axis=1, keepdims=True)
        cols_i.append(a)
        cols_w.append(mk)
        pw = jnp.where(iota == a, -1.0, pw)
    zi = jnp.zeros_like(cols_i[0])
    zw = jnp.zeros_like(cols_w[0])
    idx_full = jnp.concatenate(cols_i + [zi], axis=1)
    w_full = jnp.concatenate(cols_w + [zw], axis=1)
    wsum = cols_w[0] + cols_w[1] + cols_w[2]
    idx_ref[...] = idx_full
    w_ref[...] = w_full / wsum

    mid = _gelu(jnp.dot(x, w1_ref[...], preferred_element_type=_F32) + b1_ref[...])
    sh_ref[...] = jnp.dot(mid, w2_ref[...], preferred_element_type=_F32) + b2_ref[...]


def _router_shared(x, router_W, router_b, sh_W1, sh_b1, sh_W2, sh_b2):
    grid = (B // TB1,)
    return pl.pallas_call(
        _router_shared_body,
        grid=grid,
        in_specs=[
            pl.BlockSpec((TB1, D), lambda i: (i, 0)),
            pl.BlockSpec((D, E), lambda i: (0, 0)),
            pl.BlockSpec((1, E), lambda i: (0, 0)),
            pl.BlockSpec((D, H), lambda i: (0, 0)),
            pl.BlockSpec((1, H), lambda i: (0, 0)),
            pl.BlockSpec((H, O), lambda i: (0, 0)),
            pl.BlockSpec((1, O), lambda i: (0, 0)),
        ],
        out_specs=[
            pl.BlockSpec((TB1, 4), lambda i: (i, 0)),
            pl.BlockSpec((TB1, 4), lambda i: (i, 0)),
            pl.BlockSpec((TB1, O), lambda i: (i, 0)),
            pl.BlockSpec((8, 128), lambda i: (0, 0)),
            pl.BlockSpec((TB1, D // 2), lambda i: (i, 0)),
        ],
        out_shape=[
            jax.ShapeDtypeStruct((B, 4), _I32),
            jax.ShapeDtypeStruct((B, 4), _F32),
            jax.ShapeDtypeStruct((B, O), _F32),
            jax.ShapeDtypeStruct((8, 128), _F32),
            jax.ShapeDtypeStruct((B, D // 2), _F32),
        ],
    )(x, router_W, router_b.reshape(1, E), sh_W1, sh_b1.reshape(1, H),
      sh_W2, sh_b2.reshape(1, O))


@functools.cache
def _make_sc_gather(n_rows, width, chunk, nbuf=4, dtype=_F32):
    rpw = n_rows // NW
    nch = rpw // chunk
    mesh = plsc.VectorSubcoreMesh(core_axis_name="c", subcore_axis_name="s")
    buf_types = [pltpu.VMEM((chunk, width), dtype) for _ in range(nbuf)]
    gsem_types = [pltpu.SemaphoreType.DMA for _ in range(nbuf)]
    ssem_types = [pltpu.SemaphoreType.DMA for _ in range(nbuf)]

    @functools.partial(
        pl.kernel,
        mesh=mesh,
        out_type=jax.ShapeDtypeStruct((n_rows, width), dtype),
        scratch_types=[pltpu.VMEM((rpw,), _I32)] + buf_types + gsem_types + ssem_types,
    )
    def sc_gather(src_hbm, idx_hbm, out_hbm, idx_v, *rest):
        bufs = rest[:nbuf]
        gsems = rest[nbuf:2 * nbuf]
        ssems = rest[2 * nbuf:]
        wid = lax.axis_index("s") * 2 + lax.axis_index("c")
        base = wid * rpw
        pltpu.sync_copy(idx_hbm.at[pl.ds(base, rpw)], idx_v)

        def gather(c):
            return pltpu.async_copy(
                src_hbm.at[idx_v.at[pl.ds(c * chunk, chunk)]],
                bufs[c % nbuf], gsems[c % nbuf])

        g_h = [None] * nch
        s_h = [None] * nch
        for c in range(min(nbuf, nch)):
            g_h[c] = gather(c)
        for c in range(nch):
            g_h[c].wait()
            s_h[c] = pltpu.async_copy(
                bufs[c % nbuf], out_hbm.at[pl.ds(base + c * chunk, chunk)],
                ssems[c % nbuf])
            if c + nbuf < nch:
                s_h[c].wait()
                g_h[c + nbuf] = gather(c + nbuf)
        for c in range(max(0, nch - nbuf), nch):
            s_h[c].wait()

    return sc_gather


def _expert_mlp_body(te_ref, xs_ref, w1_ref, b1_ref, w2_ref, b2_ref, out_ref):
    v = lax.bitcast_convert_type(xs_ref[...], jnp.uint32)
    xa = lax.bitcast_convert_type(v & jnp.uint32(0xFFFF0000), _F32)
    xb = lax.bitcast_convert_type(v << 16, _F32)
    xs = jnp.concatenate([xa, xb], axis=1)
    h = _gelu(jnp.dot(xs, w1_ref[0], preferred_element_type=_F32)
              + b1_ref[0])
    out_ref[...] = jnp.dot(h, w2_ref[0], preferred_element_type=_F32) + b2_ref[0]


def _expert_mlp(x_sorted, tile_expert, ex_W1, ex_b1, ex_W2, ex_b2):
    grid_spec = pltpu.PrefetchScalarGridSpec(
        num_scalar_prefetch=1,
        grid=(NT,),
        in_specs=[
            pl.BlockSpec((T, D // 2), lambda t, te: (t, 0)),
            pl.BlockSpec((1, D, H), lambda t, te: (te[t], 0, 0)),
            pl.BlockSpec((1, 1, H), lambda t, te: (te[t], 0, 0)),
            pl.BlockSpec((1, H, O2), lambda t, te: (te[t], 0, 0)),
            pl.BlockSpec((1, 1, O2), lambda t, te: (te[t], 0, 0)),
        ],
        out_specs=pl.BlockSpec((T, O2), lambda t, te: (t, 0)),
    )
    w2p = jnp.pad(ex_W2, ((0, 0), (0, 0), (0, O2 - O)))
    b2p = jnp.pad(ex_b2, ((0, 0), (0, O2 - O))).reshape(E, 1, O2)
    return pl.pallas_call(
        _expert_mlp_body,
        grid_spec=grid_spec,
        out_shape=jax.ShapeDtypeStruct((P, O2), _F32),
    )(tile_expert, x_sorted, ex_W1, ex_b1.reshape(E, 1, H), w2p, b2p)


def _heads_body(sh_ref, r3_ref, w_ref, qw_ref, qb_ref, lw_ref, lb_ref,
                quad_ref, ll_ref, comb_ref):
    j = pl.program_id(1)

    @pl.when(j == 0)
    def _():
        r3 = r3_ref[...]
        w = w_ref[...]
        routed = (r3[:, 0, :] * w[:, 0:1]
                  + r3[:, 1, :] * w[:, 1:2]
                  + r3[:, 2, :] * w[:, 2:3])
        comb = sh_ref[...] + routed[:, 0:O]
        comb_ref[...] = comb
        ll_ref[...] = jnp.dot(comb, lw_ref[...], preferred_element_type=_F32) + lb_ref[...]

    quad_ref[...] = (jnp.dot(comb_ref[...], qw_ref[...], preferred_element_type=_F32)
                     + qb_ref[...])


def _heads(shared, routed3, topk_w, qt_W, qt_b, ll_W, ll_b):
    grid = (B // TB2, pl.cdiv(Q, TQ))
    return pl.pallas_call(
        _heads_body,
        grid=grid,
        in_specs=[
            pl.BlockSpec((TB2, O), lambda i, j: (i, 0)),
            pl.BlockSpec((TB2, K, O2), lambda i, j: (i, 0, 0)),
            pl.BlockSpec((TB2, 4), lambda i, j: (i, 0)),
            pl.BlockSpec((O, TQ), lambda i, j: (0, j)),
            pl.BlockSpec((1, TQ), lambda i, j: (0, j)),
            pl.BlockSpec((O, 2), lambda i, j: (0, 0)),
            pl.BlockSpec((1, 2), lambda i, j: (0, 0)),
        ],
        out_specs=[
            pl.BlockSpec((TB2, TQ), lambda i, j: (i, j)),
            pl.BlockSpec((TB2, 2), lambda i, j: (i, 0)),
        ],
        out_shape=[
            jax.ShapeDtypeStruct((B, Q), _F32),
            jax.ShapeDtypeStruct((B, 2), _F32),
        ],
        scratch_shapes=[pltpu.VMEM((TB2, O), _F32)],
    )(shared, routed3, topk_w, qt_W, qt_b.reshape(1, Q), ll_W, ll_b.reshape(1, 2))


def _dispatch_meta(topk_idx):
    mask_be = (topk_idx[:, :, None] == jnp.arange(E)[None, None, :]).any(axis=1)
    mask_be = mask_be.astype(_I32)
    csum = jnp.cumsum(mask_be, axis=0)
    rank_be = csum - mask_be
    counts = csum[-1]
    tiles_e = (counts + T - 1) // T
    tile_start = jnp.concatenate([jnp.zeros((1,), _I32),
                                  jnp.cumsum(tiles_e).astype(_I32)])
    pstart = tile_start[:E] * T
    dest_be = pstart[None, :] + rank_be
    dest3 = jnp.take_along_axis(dest_be, topk_idx, axis=1).astype(_I32)
    slot_token = jnp.zeros((P,), _I32).at[dest3.reshape(-1)].set(
        jnp.repeat(jnp.arange(B, dtype=_I32), K), mode="drop")
    tile_expert = jnp.searchsorted(
        jnp.cumsum(tiles_e), jnp.arange(NT, dtype=_I32), side="right")
    tile_expert = jnp.minimum(tile_expert, E - 1).astype(_I32)
    return dest3, slot_token, tile_expert, counts


def kernel(x, router_W, router_b, sh_W1, sh_b1, sh_W2, sh_b2,
           ex_W1, ex_b1, ex_W2, ex_b2, ll_W, ll_b, qt_W, qt_b):
    topk_idx8, topk_w, shared, stats, x_bf = _router_shared(
        x, router_W, router_b, sh_W1, sh_b1, sh_W2, sh_b2)
    topk_idx = topk_idx8[:, :K]
    dest3, slot_token, tile_expert, counts = _dispatch_meta(topk_idx)

    x_sorted = _make_sc_gather(P, D // 2, CH)(x_bf, slot_token)
    out_sorted = _expert_mlp(x_sorted, tile_expert, ex_W1, ex_b1, ex_W2, ex_b2)
    routed3 = _make_sc_gather(B * K, O2, BKW, nbuf=1)(
        out_sorted, dest3.reshape(-1)).reshape(B, K, O2)

    quad, ll = _heads(shared, routed3, topk_w, qt_W, qt_b, ll_W, ll_b)

    p_mean = stats[0, 0:E] / B
    f_i = (E / (K * B)) * counts.astype(_F32)
    balance_loss = jnp.sum(f_i * p_mean) * 0.001
    return ll[:, 0], ll[:, 1], quad, balance_loss
